# Initial kernel scaffold; baseline (speedup 1.0000x reference)
import jax
import jax.numpy as jnp
from jax.experimental import pallas as pl


def kernel(x, edge_index, Wl0, bl0, Wr0, Wl1, bl1, Wr1, Wl2, bl2, Wr2):
    raise NotImplementedError("write your pallas kernel here")


# speedup vs baseline: 3.3749x; 3.3749x over previous
import functools

import jax
import jax.numpy as jnp
from jax import lax
from jax.experimental import pallas as pl
from jax.experimental.pallas import tpu as pltpu
from jax.experimental.pallas import tpu_sc as plsc

N = 10000
E = 160000
D = 256
DH = 128
TILES = 16
CHUNK = 128
EP = 163840
CHUNKS_PER_TILE = EP // (TILES * CHUNK)
NP = 10240
ROWS_PER_TILE = NP // TILES

_mesh = plsc.VectorSubcoreMesh(core_axis_name="c", subcore_axis_name="s")


def _sc_agg_body(compute_deg, args):
    if compute_deg:
        (x0, x1, src2d, dst2d, z2d, z1d,
         agg0_out, agg1_out, deg_out,
         agg_s, deg_s, src_v, dst_v, rows_v, ones_v, sem) = args
    else:
        (x0, x1, src2d, dst2d, z2d, z1d,
         agg0_out, agg1_out,
         agg_s, src_v, dst_v, rows_v, sem) = args

    c = lax.axis_index("c")
    t = lax.axis_index("s")
    base_row = t * ROWS_PER_TILE

    pltpu.sync_copy(z2d, agg_s.at[pl.ds(base_row, ROWS_PER_TILE)])
    if compute_deg:
        pltpu.sync_copy(z1d, deg_s.at[pl.ds(base_row, ROWS_PER_TILE)])
        for i in range(8):
            ones_v[pl.ds(i * 16, 16)] = jnp.ones((16,), jnp.float32)

    pltpu.sync_copy(src2d.at[pl.ds(t * CHUNKS_PER_TILE, CHUNKS_PER_TILE)], src_v)
    pltpu.sync_copy(dst2d.at[pl.ds(t * CHUNKS_PER_TILE, CHUNKS_PER_TILE)], dst_v)

    plsc.subcore_barrier()

    def run(xref, with_deg):
        def body(j, carry):
            pltpu.async_copy(xref.at[src_v.at[j]], rows_v, sem).wait()
            pltpu.sync_copy(rows_v, agg_s.at[dst_v.at[j]], add=True)
            if with_deg:
                pltpu.sync_copy(ones_v, deg_s.at[dst_v.at[j]], add=True)
            return carry
        lax.fori_loop(0, CHUNKS_PER_TILE, body, 0)

    @pl.when(c == 0)
    def _():
        run(x0, compute_deg)

    @pl.when(c == 1)
    def _():
        run(x1, False)

    plsc.subcore_barrier()

    @pl.when(c == 0)
    def _():
        pltpu.sync_copy(agg_s.at[pl.ds(base_row, ROWS_PER_TILE)],
                        agg0_out.at[pl.ds(base_row, ROWS_PER_TILE)])
        if compute_deg:
            pltpu.sync_copy(deg_s.at[pl.ds(base_row, ROWS_PER_TILE)],
                            deg_out.at[pl.ds(base_row, ROWS_PER_TILE)])

    @pl.when(c == 1)
    def _():
        pltpu.sync_copy(agg_s.at[pl.ds(base_row, ROWS_PER_TILE)],
                        agg1_out.at[pl.ds(base_row, ROWS_PER_TILE)])


def _make_sc_agg(compute_deg):
    outs = [jax.ShapeDtypeStruct((NP, DH), jnp.float32),
            jax.ShapeDtypeStruct((NP, DH), jnp.float32)]
    scratch = [pltpu.VMEM_SHARED((NP, DH), jnp.float32)]
    if compute_deg:
        outs.append(jax.ShapeDtypeStruct((NP,), jnp.float32))
        scratch.append(pltpu.VMEM_SHARED((NP,), jnp.float32))
    scratch += [
        pltpu.VMEM((CHUNKS_PER_TILE, CHUNK), jnp.int32),
        pltpu.VMEM((CHUNKS_PER_TILE, CHUNK), jnp.int32),
        pltpu.VMEM((CHUNK, DH), jnp.float32),
    ]
    if compute_deg:
        scratch.append(pltpu.VMEM((CHUNK,), jnp.float32))
    scratch.append(pltpu.SemaphoreType.DMA)

    def body(*args):
        _sc_agg_body(compute_deg, args)

    return pl.kernel(body, out_type=tuple(outs), mesh=_mesh,
                     scratch_types=scratch)


_sc_agg_deg = _make_sc_agg(True)
_sc_agg = _make_sc_agg(False)


def _tc_layer_body(relu, agg0, agg1, deg, h, wl, wr, bl, out):
    rdeg = 1.0 / jnp.maximum(deg[...], 1.0)
    m = jnp.dot(agg0[...] * rdeg, wl[0:DH, :],
                preferred_element_type=jnp.float32)
    m = m + jnp.dot(agg1[...] * rdeg, wl[DH:D, :],
                    preferred_element_type=jnp.float32)
    m = m + jnp.dot(h[...], wr[...], preferred_element_type=jnp.float32)
    m = m + bl[...]
    out[...] = jnp.maximum(m, 0.0) if relu else m


def _make_tc_layer(relu):
    R = 2000
    grid = (N // R,)
    return pl.pallas_call(
        functools.partial(_tc_layer_body, relu),
        grid=grid,
        in_specs=[
            pl.BlockSpec((R, DH), lambda i: (i, 0)),
            pl.BlockSpec((R, DH), lambda i: (i, 0)),
            pl.BlockSpec((R, 1), lambda i: (i, 0)),
            pl.BlockSpec((R, D), lambda i: (i, 0)),
            pl.BlockSpec((D, D), lambda i: (0, 0)),
            pl.BlockSpec((D, D), lambda i: (0, 0)),
            pl.BlockSpec((1, D), lambda i: (0, 0)),
        ],
        out_specs=pl.BlockSpec((R, D), lambda i: (i, 0)),
        out_shape=jax.ShapeDtypeStruct((N, D), jnp.float32),
    )


_tc_layer_relu = _make_tc_layer(True)
_tc_layer_last = _make_tc_layer(False)


def kernel(x, edge_index, Wl0, bl0, Wr0, Wl1, bl1, Wr1, Wl2, bl2, Wr2):
    src = edge_index[0]
    dst = edge_index[1]
    src2d = jnp.concatenate(
        [src, jnp.zeros((EP - E,), jnp.int32)]).reshape(EP // CHUNK, CHUNK)
    dst2d = jnp.concatenate(
        [dst, jnp.full((EP - E,), N, jnp.int32)]).reshape(EP // CHUNK, CHUNK)
    z2d = jnp.zeros((ROWS_PER_TILE, DH), jnp.float32)
    z1d = jnp.zeros((ROWS_PER_TILE,), jnp.float32)

    h = x
    deg2 = None
    layers = [(Wl0, bl0, Wr0, True), (Wl1, bl1, Wr1, True),
              (Wl2, bl2, Wr2, False)]
    for li, (Wl, bl, Wr, relu) in enumerate(layers):
        x0 = h[:, :DH]
        x1 = h[:, DH:]
        if li == 0:
            agg0, agg1, degf = _sc_agg_deg(x0, x1, src2d, dst2d, z2d, z1d)
            deg2 = degf[:N].reshape(N, 1)
        else:
            agg0, agg1 = _sc_agg(x0, x1, src2d, dst2d, z2d, z1d)
        tc = _tc_layer_relu if relu else _tc_layer_last
        h = tc(agg0, agg1, deg2, h, Wl, Wr, bl.reshape(1, D))
    return h


# speedup vs baseline: 4.0344x; 1.1954x over previous
import functools

import jax
import jax.numpy as jnp
from jax import lax
from jax.experimental import pallas as pl
from jax.experimental.pallas import tpu as pltpu
from jax.experimental.pallas import tpu_sc as plsc

N = 10000
E = 160000
D = 256
DH = 128
TILES = 16
CHUNK = 128
EP = 163840
CHUNKS_PER_TILE = EP // (TILES * CHUNK)
NP = 10240
ROWS_PER_TILE = NP // TILES

_mesh = plsc.VectorSubcoreMesh(core_axis_name="c", subcore_axis_name="s")


NBUF = 2
AHEAD = 1
PHASES = 2
PCH = CHUNKS_PER_TILE // PHASES


def _sc_agg_body(compute_deg, args):
    if compute_deg:
        (x0, x1, src2d, dst2d, z2d, z1d,
         agg0_out, agg1_out, deg_out,
         agg_s, deg_s, src_v, dst_v) = args[:13]
        rows = args[13:13 + NBUF]
        ones_v = args[13 + NBUF]
        gsem = args[14 + NBUF:14 + 2 * NBUF]
        ssem = args[14 + 2 * NBUF:14 + 3 * NBUF]
        dsem = args[14 + 3 * NBUF:14 + 4 * NBUF]
    else:
        (x0, x1, src2d, dst2d, z2d, z1d,
         agg0_out, agg1_out,
         agg_s, src_v, dst_v) = args[:11]
        rows = args[11:11 + NBUF]
        gsem = args[11 + NBUF:11 + 2 * NBUF]
        ssem = args[11 + 2 * NBUF:11 + 3 * NBUF]
        dsem = None

    c = lax.axis_index("c")
    t = lax.axis_index("s")
    base_row = t * ROWS_PER_TILE

    pltpu.sync_copy(z2d, agg_s.at[pl.ds(base_row, ROWS_PER_TILE)])
    if compute_deg:
        pltpu.sync_copy(z1d, deg_s.at[pl.ds(base_row, ROWS_PER_TILE)])
        for i in range(8):
            ones_v[pl.ds(i * 16, 16)] = jnp.ones((16,), jnp.float32)

    plsc.subcore_barrier()

    def run(xref, with_deg):
        def g_start(j, b):
            pltpu.async_copy(xref.at[src_v.at[j]], rows[b], gsem[b])

        def g_wait(b):
            pltpu.make_async_copy(xref.at[src_v.at[0]], rows[b],
                                  gsem[b]).wait()

        def s_start(j, b):
            pltpu.async_copy(rows[b], agg_s.at[dst_v.at[j]], ssem[b],
                             add=True)

        def s_wait(b):
            pltpu.make_async_copy(rows[b], agg_s.at[dst_v.at[0]],
                                  ssem[b]).wait()

        def d_start(j, b):
            pltpu.async_copy(ones_v, deg_s.at[dst_v.at[j]], dsem[b],
                             add=True)

        def d_wait(b):
            pltpu.make_async_copy(ones_v, deg_s.at[dst_v.at[0]],
                                  dsem[b]).wait()

        for p in range(PHASES):
            base = t * CHUNKS_PER_TILE + p * PCH
            pltpu.sync_copy(src2d.at[pl.ds(base, PCH)], src_v)
            pltpu.sync_copy(dst2d.at[pl.ds(base, PCH)], dst_v)

            for j in range(AHEAD):
                g_start(j, j)

            def body(i, carry):
                for b in range(NBUF):
                    j = NBUF * i + b
                    b2 = (b + AHEAD) % NBUF

                    @pl.when(j >= AHEAD)
                    def _():
                        s_wait(b2)
                        if with_deg:
                            d_wait(b2)

                    @pl.when(j < PCH - AHEAD)
                    def _():
                        g_start(j + AHEAD, b2)

                    g_wait(b)
                    s_start(j, b)
                    if with_deg:
                        d_start(j, b)
                return carry

            lax.fori_loop(0, PCH // NBUF, body, 0)

            for j in range(PCH - AHEAD, PCH):
                s_wait(j % NBUF)
                if with_deg:
                    d_wait(j % NBUF)

    @pl.when(c == 0)
    def _():
        run(x0, compute_deg)

    @pl.when(c == 1)
    def _():
        run(x1, False)

    plsc.subcore_barrier()

    @pl.when(c == 0)
    def _():
        pltpu.sync_copy(agg_s.at[pl.ds(base_row, ROWS_PER_TILE)],
                        agg0_out.at[pl.ds(base_row, ROWS_PER_TILE)])
        if compute_deg:
            pltpu.sync_copy(deg_s.at[pl.ds(base_row, ROWS_PER_TILE)],
                            deg_out.at[pl.ds(base_row, ROWS_PER_TILE)])

    @pl.when(c == 1)
    def _():
        pltpu.sync_copy(agg_s.at[pl.ds(base_row, ROWS_PER_TILE)],
                        agg1_out.at[pl.ds(base_row, ROWS_PER_TILE)])


def _make_sc_agg(compute_deg):
    outs = [jax.ShapeDtypeStruct((NP, DH), jnp.float32),
            jax.ShapeDtypeStruct((NP, DH), jnp.float32)]
    scratch = [pltpu.VMEM_SHARED((NP, DH), jnp.float32)]
    if compute_deg:
        outs.append(jax.ShapeDtypeStruct((NP,), jnp.float32))
        scratch.append(pltpu.VMEM_SHARED((NP,), jnp.float32))
    scratch += [
        pltpu.VMEM((PCH, CHUNK), jnp.int32),
        pltpu.VMEM((PCH, CHUNK), jnp.int32),
    ]
    scratch += [pltpu.VMEM((CHUNK, DH), jnp.float32)
                for _ in range(NBUF)]
    if compute_deg:
        scratch.append(pltpu.VMEM((CHUNK,), jnp.float32))
    scratch += [pltpu.SemaphoreType.DMA] * (2 * NBUF)
    if compute_deg:
        scratch += [pltpu.SemaphoreType.DMA] * NBUF

    def body(*args):
        _sc_agg_body(compute_deg, args)

    return pl.kernel(body, out_type=tuple(outs), mesh=_mesh,
                     scratch_types=scratch)


_sc_agg_deg = _make_sc_agg(True)
_sc_agg = _make_sc_agg(False)


def _tc_layer_body(relu, agg0, agg1, deg, h, wl, wr, bl, out):
    rdeg = 1.0 / jnp.maximum(deg[...], 1.0)
    m = jnp.dot(agg0[...] * rdeg, wl[0:DH, :],
                preferred_element_type=jnp.float32)
    m = m + jnp.dot(agg1[...] * rdeg, wl[DH:D, :],
                    preferred_element_type=jnp.float32)
    m = m + jnp.dot(h[...], wr[...], preferred_element_type=jnp.float32)
    m = m + bl[...]
    out[...] = jnp.maximum(m, 0.0) if relu else m


def _make_tc_layer(relu):
    R = 2000
    grid = (N // R,)
    return pl.pallas_call(
        functools.partial(_tc_layer_body, relu),
        grid=grid,
        in_specs=[
            pl.BlockSpec((R, DH), lambda i: (i, 0)),
            pl.BlockSpec((R, DH), lambda i: (i, 0)),
            pl.BlockSpec((R, 1), lambda i: (i, 0)),
            pl.BlockSpec((R, D), lambda i: (i, 0)),
            pl.BlockSpec((D, D), lambda i: (0, 0)),
            pl.BlockSpec((D, D), lambda i: (0, 0)),
            pl.BlockSpec((1, D), lambda i: (0, 0)),
        ],
        out_specs=pl.BlockSpec((R, D), lambda i: (i, 0)),
        out_shape=jax.ShapeDtypeStruct((N, D), jnp.float32),
    )


_tc_layer_relu = _make_tc_layer(True)
_tc_layer_last = _make_tc_layer(False)


def kernel(x, edge_index, Wl0, bl0, Wr0, Wl1, bl1, Wr1, Wl2, bl2, Wr2):
    src = edge_index[0]
    dst = edge_index[1]
    src2d = jnp.concatenate(
        [src, jnp.zeros((EP - E,), jnp.int32)]).reshape(EP // CHUNK, CHUNK)
    dst2d = jnp.concatenate(
        [dst, jnp.full((EP - E,), N, jnp.int32)]).reshape(EP // CHUNK, CHUNK)
    z2d = jnp.zeros((ROWS_PER_TILE, DH), jnp.float32)
    z1d = jnp.zeros((ROWS_PER_TILE,), jnp.float32)

    h = x
    deg2 = None
    layers = [(Wl0, bl0, Wr0, True), (Wl1, bl1, Wr1, True),
              (Wl2, bl2, Wr2, False)]
    for li, (Wl, bl, Wr, relu) in enumerate(layers):
        x0 = h[:, :DH]
        x1 = h[:, DH:]
        if li == 0:
            agg0, agg1, degf = _sc_agg_deg(x0, x1, src2d, dst2d, z2d, z1d)
            deg2 = degf[:N].reshape(N, 1)
        else:
            agg0, agg1 = _sc_agg(x0, x1, src2d, dst2d, z2d, z1d)
        tc = _tc_layer_relu if relu else _tc_layer_last
        h = tc(agg0, agg1, deg2, h, Wl, Wr, bl.reshape(1, D))
    return h


# speedup vs baseline: 5.8336x; 1.4460x over previous
import functools

import jax
import jax.numpy as jnp
from jax import lax
from jax.experimental import pallas as pl
from jax.experimental.pallas import tpu as pltpu
from jax.experimental.pallas import tpu_sc as plsc

N = 10000
E = 160000
D = 256
DH = 128
TILES = 16
NP = 10240
ROWS_PER_TILE = NP // TILES
EP = 163840
EPT = EP // TILES
NB = 4
NPB = NP // NB
SROWS = NPB // TILES
CAP = 3072
AC = 64
NCH = CAP // AC
NBUF = 2
AHEAD = 1

_mesh = plsc.VectorSubcoreMesh(core_axis_name="c", subcore_axis_name="s")


def _part_body(src2d, dst2d, z1d, junk_s, junk_d, psrc_out, pdst_out,
               deg_out,
               deg_s, src_v, dst_v, pb0s, pb0d, pb1s, pb1d, pb2s, pb2d,
               pb3s, pb3d, ones_v):
    c = lax.axis_index("c")
    t = lax.axis_index("s")
    pbs = (pb0s, pb1s, pb2s, pb3s)
    pbd = (pb0d, pb1d, pb2d, pb3d)

    @pl.when(c == 0)
    def _():
        pltpu.sync_copy(z1d, deg_s.at[pl.ds(t * ROWS_PER_TILE,
                                            ROWS_PER_TILE)])
        pltpu.sync_copy(src2d.at[pl.ds(t * (EPT // 128), EPT // 128)], src_v)
        pltpu.sync_copy(dst2d.at[pl.ds(t * (EPT // 128), EPT // 128)], dst_v)
        for k in range(NB):
            pltpu.sync_copy(junk_s, pbs[k])
            pltpu.sync_copy(junk_d, pbd[k])
        for i in range(8):
            ones_v[pl.ds(i * 16, 16)] = jnp.ones((16,), jnp.float32)
        plsc.subcore_barrier()

        def row(r, off):
            pltpu.sync_copy(ones_v, deg_s.at[dst_v.at[r]], add=True)
            for g in range(8):
                sl = pl.ds(g * 16, 16)
                s = src_v[r, sl]
                d = dst_v[r, sl]
                new = []
                for k in range(NB):
                    lo = k * NPB
                    m = (s >= lo) & (s < lo + NPB)
                    cs = plsc.cumsum(m.astype(jnp.int32))
                    idxv = off[k] + cs - 1
                    plsc.store_scatter(pbs[k], (idxv,), s - lo, mask=m)
                    plsc.store_scatter(pbd[k], (idxv,), d, mask=m)
                    new.append(off[k] + jnp.max(cs))
                off = tuple(new)
            return off

        lax.fori_loop(0, EPT // 128, row, (0, 0, 0, 0))
        plsc.subcore_barrier()

        for k in range(NB):
            pltpu.sync_copy(pbs[k], psrc_out.at[k, t])
            pltpu.sync_copy(pbd[k], pdst_out.at[k, t])
        pltpu.sync_copy(deg_s.at[pl.ds(t * ROWS_PER_TILE, ROWS_PER_TILE)],
                        deg_out.at[pl.ds(t * ROWS_PER_TILE, ROWS_PER_TILE)])


_partition = pl.kernel(
    _part_body,
    out_type=(jax.ShapeDtypeStruct((NB, TILES, CAP), jnp.int32),
              jax.ShapeDtypeStruct((NB, TILES, CAP), jnp.int32),
              jax.ShapeDtypeStruct((NP,), jnp.float32)),
    mesh=_mesh,
    scratch_types=[
        pltpu.VMEM_SHARED((NP,), jnp.float32),
        pltpu.VMEM((EPT // 128, 128), jnp.int32),
        pltpu.VMEM((EPT // 128, 128), jnp.int32),
    ] + [pltpu.VMEM((CAP,), jnp.int32) for _ in range(2 * NB)]
    + [pltpu.VMEM((128,), jnp.float32)],
    compiler_params=pltpu.CompilerParams(needs_layout_passes=False),
)


def _agg_body(x0, x1, psrc, pdst, z128,
              agg0_out, agg1_out,
              x_s, agg_s, src_v, dst_v, rows0, rows1, gsem0, gsem1,
              ssem0, ssem1):
    c = lax.axis_index("c")
    t = lax.axis_index("s")
    base_row = t * ROWS_PER_TILE
    rows = (rows0, rows1)
    gsem = (gsem0, gsem1)
    ssem = (ssem0, ssem1)

    def run(xref, agg_out):
        pltpu.sync_copy(z128, agg_s.at[pl.ds(base_row, ROWS_PER_TILE)])

        def g_start(j, b):
            pltpu.async_copy(x_s.at[src_v.at[j]], rows[b], gsem[b])

        def g_wait(b):
            pltpu.make_async_copy(x_s.at[src_v.at[0]], rows[b],
                                  gsem[b]).wait()

        def s_start(j, b):
            pltpu.async_copy(rows[b], agg_s.at[dst_v.at[j]], ssem[b],
                             add=True)

        def s_wait(b):
            pltpu.make_async_copy(rows[b], agg_s.at[dst_v.at[0]],
                                  ssem[b]).wait()

        for k in range(NB):
            pltpu.sync_copy(
                xref.at[pl.ds(k * NPB + t * SROWS, SROWS)],
                x_s.at[pl.ds(t * SROWS, SROWS)])
            pltpu.sync_copy(psrc.at[k, t], src_v)
            pltpu.sync_copy(pdst.at[k, t], dst_v)
            plsc.subcore_barrier()

            for j in range(AHEAD):
                g_start(j, j)

            def body(i, carry):
                for b in range(NBUF):
                    j = NBUF * i + b
                    b2 = (b + AHEAD) % NBUF

                    @pl.when(j >= AHEAD)
                    def _():
                        s_wait(b2)

                    @pl.when(j < NCH - AHEAD)
                    def _():
                        g_start(j + AHEAD, b2)

                    g_wait(b)
                    s_start(j, b)
                return carry

            lax.fori_loop(0, NCH // NBUF, body, 0)

            for j in range(NCH - AHEAD, NCH):
                s_wait(j % NBUF)

            plsc.subcore_barrier()

        pltpu.sync_copy(agg_s.at[pl.ds(base_row, ROWS_PER_TILE)],
                        agg_out.at[pl.ds(base_row, ROWS_PER_TILE)])

    @pl.when(c == 0)
    def _():
        run(x0, agg0_out)

    @pl.when(c == 1)
    def _():
        run(x1, agg1_out)


_sc_agg = pl.kernel(
    _agg_body,
    out_type=(jax.ShapeDtypeStruct((NP, DH), jnp.float32),
              jax.ShapeDtypeStruct((NP, DH), jnp.float32)),
    mesh=_mesh,
    scratch_types=[
        pltpu.VMEM_SHARED((NPB, DH), jnp.float32),
        pltpu.VMEM_SHARED((NP, DH), jnp.float32),
        pltpu.VMEM((NCH, AC), jnp.int32),
        pltpu.VMEM((NCH, AC), jnp.int32),
        pltpu.VMEM((AC, DH), jnp.float32),
        pltpu.VMEM((AC, DH), jnp.float32),
        pltpu.SemaphoreType.DMA, pltpu.SemaphoreType.DMA,
        pltpu.SemaphoreType.DMA, pltpu.SemaphoreType.DMA,
    ],
)


def _tc_layer_body(relu, agg0, agg1, deg, h, wl, wr, bl, out):
    rdeg = 1.0 / jnp.maximum(deg[...], 1.0)
    m = jnp.dot(agg0[...], wl[0:DH, :], preferred_element_type=jnp.float32)
    m = m + jnp.dot(agg1[...], wl[DH:D, :],
                    preferred_element_type=jnp.float32)
    m = m * rdeg
    m = m + jnp.dot(h[...], wr[...], preferred_element_type=jnp.float32)
    m = m + bl[...]
    out[...] = jnp.maximum(m, 0.0) if relu else m


def _make_tc_layer(relu):
    R = 2000
    grid = (N // R,)
    return pl.pallas_call(
        functools.partial(_tc_layer_body, relu),
        grid=grid,
        in_specs=[
            pl.BlockSpec((R, DH), lambda i: (i, 0)),
            pl.BlockSpec((R, DH), lambda i: (i, 0)),
            pl.BlockSpec((R, 1), lambda i: (i, 0)),
            pl.BlockSpec((R, D), lambda i: (i, 0)),
            pl.BlockSpec((D, D), lambda i: (0, 0)),
            pl.BlockSpec((D, D), lambda i: (0, 0)),
            pl.BlockSpec((1, D), lambda i: (0, 0)),
        ],
        out_specs=pl.BlockSpec((R, D), lambda i: (i, 0)),
        out_shape=jax.ShapeDtypeStruct((N, D), jnp.float32),
    )


_tc_layer_relu = _make_tc_layer(True)
_tc_layer_last = _make_tc_layer(False)


def kernel(x, edge_index, Wl0, bl0, Wr0, Wl1, bl1, Wr1, Wl2, bl2, Wr2):
    src = edge_index[0]
    dst = edge_index[1]
    pad_src = (jnp.arange(EP - E, dtype=jnp.int32) % NB) * NPB
    src2d = jnp.concatenate([src, pad_src]).reshape(EP // 128, 128)
    dst2d = jnp.concatenate(
        [dst, jnp.full((EP - E,), N, jnp.int32)]).reshape(EP // 128, 128)
    z1d = jnp.zeros((ROWS_PER_TILE,), jnp.float32)
    z128 = jnp.zeros((ROWS_PER_TILE, DH), jnp.float32)

    junk_s = jnp.zeros((CAP,), jnp.int32)
    junk_d = jnp.full((CAP,), N, jnp.int32)
    psrc, pdst, degf = _partition(src2d, dst2d, z1d, junk_s, junk_d)
    psrc = psrc.reshape(NB, TILES, NCH, AC)
    pdst = pdst.reshape(NB, TILES, NCH, AC)
    deg2 = degf[:N].reshape(N, 1)

    h = x
    layers = [(Wl0, bl0, Wr0, True), (Wl1, bl1, Wr1, True),
              (Wl2, bl2, Wr2, False)]
    for Wl, bl, Wr, relu in layers:
        hp = jnp.pad(h, ((0, NP - N), (0, 0)))
        a0, a1 = _sc_agg(hp[:, :DH], hp[:, DH:], psrc, pdst, z128)
        tc = _tc_layer_relu if relu else _tc_layer_last
        h = tc(a0, a1, deg2, h, Wl, Wr, bl.reshape(1, D))
    return h


# speedup vs baseline: 5.9705x; 1.0235x over previous
import functools

import jax
import jax.numpy as jnp
from jax import lax
from jax.experimental import pallas as pl
from jax.experimental.pallas import tpu as pltpu
from jax.experimental.pallas import tpu_sc as plsc

N = 10000
E = 160000
D = 256
DH = 128
TILES = 16
NP = 10240
ROWS_PER_TILE = NP // TILES
EP = 163840
EPT = EP // TILES
NB = 4
NPB = NP // NB
SROWS = NPB // TILES
CAP = 2944
AC = 64
NCH = CAP // AC
NBUF = 2
AHEAD = 1

_mesh = plsc.VectorSubcoreMesh(core_axis_name="c", subcore_axis_name="s")


def _part_body(src2d, dst2d, z1d, junk_s, junk_d, psrc_out, pdst_out,
               deg_out,
               deg_s, src_v, dst_v, pb0s, pb0d, pb1s, pb1d, pb2s, pb2d,
               pb3s, pb3d, ones_v):
    c = lax.axis_index("c")
    t = lax.axis_index("s")
    pbs = (pb0s, pb1s, pb2s, pb3s)
    pbd = (pb0d, pb1d, pb2d, pb3d)

    @pl.when(c == 0)
    def _():
        pltpu.sync_copy(z1d, deg_s.at[pl.ds(t * ROWS_PER_TILE,
                                            ROWS_PER_TILE)])
        pltpu.sync_copy(src2d.at[pl.ds(t * (EPT // 128), EPT // 128)], src_v)
        pltpu.sync_copy(dst2d.at[pl.ds(t * (EPT // 128), EPT // 128)], dst_v)
        for k in range(NB):
            pltpu.sync_copy(junk_s, pbs[k])
            pltpu.sync_copy(junk_d, pbd[k])
        for i in range(8):
            ones_v[pl.ds(i * 16, 16)] = jnp.ones((16,), jnp.float32)
        plsc.subcore_barrier()

        def row(r, off):
            pltpu.sync_copy(ones_v, deg_s.at[dst_v.at[r]], add=True)
            for g in range(8):
                sl = pl.ds(g * 16, 16)
                s = src_v[r, sl]
                d = dst_v[r, sl]
                new = []
                for k in range(NB):
                    lo = k * NPB
                    m = (s >= lo) & (s < lo + NPB)
                    cs = plsc.cumsum(m.astype(jnp.int32))
                    idxv = off[k] + cs - 1
                    plsc.store_scatter(pbs[k], (idxv,), s - lo, mask=m)
                    plsc.store_scatter(pbd[k], (idxv,), d, mask=m)
                    new.append(off[k] + jnp.max(cs))
                off = tuple(new)
            return off

        lax.fori_loop(0, EPT // 128, row, (0, 0, 0, 0))
        plsc.subcore_barrier()

        for k in range(NB):
            pltpu.sync_copy(pbs[k], psrc_out.at[k, t])
            pltpu.sync_copy(pbd[k], pdst_out.at[k, t])
        pltpu.sync_copy(deg_s.at[pl.ds(t * ROWS_PER_TILE, ROWS_PER_TILE)],
                        deg_out.at[pl.ds(t * ROWS_PER_TILE, ROWS_PER_TILE)])


_partition = pl.kernel(
    _part_body,
    out_type=(jax.ShapeDtypeStruct((NB, TILES, CAP), jnp.int32),
              jax.ShapeDtypeStruct((NB, TILES, CAP), jnp.int32),
              jax.ShapeDtypeStruct((NP,), jnp.float32)),
    mesh=_mesh,
    scratch_types=[
        pltpu.VMEM_SHARED((NP,), jnp.float32),
        pltpu.VMEM((EPT // 128, 128), jnp.int32),
        pltpu.VMEM((EPT // 128, 128), jnp.int32),
    ] + [pltpu.VMEM((CAP,), jnp.int32) for _ in range(2 * NB)]
    + [pltpu.VMEM((128,), jnp.float32)],
    compiler_params=pltpu.CompilerParams(needs_layout_passes=False),
)


def _agg_body(x0, x1, psrc, pdst, z128,
              agg0_out, agg1_out,
              x_s, agg_s, src_v, dst_v, rows0, rows1, gsem0, gsem1,
              ssem0, ssem1):
    c = lax.axis_index("c")
    t = lax.axis_index("s")
    base_row = t * ROWS_PER_TILE
    rows = (rows0, rows1)
    gsem = (gsem0, gsem1)
    ssem = (ssem0, ssem1)

    def run(xref, agg_out):
        pltpu.sync_copy(z128, agg_s.at[pl.ds(base_row, ROWS_PER_TILE)])

        def g_start(j, b):
            pltpu.async_copy(x_s.at[src_v.at[j]], rows[b], gsem[b])

        def g_wait(b):
            pltpu.make_async_copy(x_s.at[src_v.at[0]], rows[b],
                                  gsem[b]).wait()

        def s_start(j, b):
            pltpu.async_copy(rows[b], agg_s.at[dst_v.at[j]], ssem[b],
                             add=True)

        def s_wait(b):
            pltpu.make_async_copy(rows[b], agg_s.at[dst_v.at[0]],
                                  ssem[b]).wait()

        for k in range(NB):
            pltpu.sync_copy(
                xref.at[pl.ds(k * NPB + t * SROWS, SROWS)],
                x_s.at[pl.ds(t * SROWS, SROWS)])
            pltpu.sync_copy(psrc.at[k, t], src_v)
            pltpu.sync_copy(pdst.at[k, t], dst_v)
            plsc.subcore_barrier()

            for j in range(AHEAD):
                g_start(j, j)

            def body(i, carry):
                for b in range(NBUF):
                    j = NBUF * i + b
                    b2 = (b + AHEAD) % NBUF

                    @pl.when(j >= AHEAD)
                    def _():
                        s_wait(b2)

                    @pl.when(j < NCH - AHEAD)
                    def _():
                        g_start(j + AHEAD, b2)

                    g_wait(b)
                    s_start(j, b)
                return carry

            lax.fori_loop(0, NCH // NBUF, body, 0)

            for j in range(NCH - AHEAD, NCH):
                s_wait(j % NBUF)

            plsc.subcore_barrier()

        pltpu.sync_copy(agg_s.at[pl.ds(base_row, ROWS_PER_TILE)],
                        agg_out.at[pl.ds(base_row, ROWS_PER_TILE)])

    @pl.when(c == 0)
    def _():
        run(x0, agg0_out)

    @pl.when(c == 1)
    def _():
        run(x1, agg1_out)


_sc_agg = pl.kernel(
    _agg_body,
    out_type=(jax.ShapeDtypeStruct((NP, DH), jnp.float32),
              jax.ShapeDtypeStruct((NP, DH), jnp.float32)),
    mesh=_mesh,
    scratch_types=[
        pltpu.VMEM_SHARED((NPB, DH), jnp.float32),
        pltpu.VMEM_SHARED((NP, DH), jnp.float32),
        pltpu.VMEM((NCH, AC), jnp.int32),
        pltpu.VMEM((NCH, AC), jnp.int32),
        pltpu.VMEM((AC, DH), jnp.float32),
        pltpu.VMEM((AC, DH), jnp.float32),
        pltpu.SemaphoreType.DMA, pltpu.SemaphoreType.DMA,
        pltpu.SemaphoreType.DMA, pltpu.SemaphoreType.DMA,
    ],
)


def _tc_root_body(h, wr, bl, out):
    out[...] = jnp.dot(h[...], wr[...],
                       preferred_element_type=jnp.float32) + bl[...]


def _tc_combine_body(relu, agg0, agg1, deg, r, wl, out):
    rdeg = 1.0 / jnp.maximum(deg[...], 1.0)
    m = jnp.dot(agg0[...], wl[0:DH, :], preferred_element_type=jnp.float32)
    m = m + jnp.dot(agg1[...], wl[DH:D, :],
                    preferred_element_type=jnp.float32)
    m = m * rdeg + r[...]
    out[...] = jnp.maximum(m, 0.0) if relu else m


_R = 2000
_tc_root = pl.pallas_call(
    _tc_root_body,
    grid=(N // _R,),
    in_specs=[
        pl.BlockSpec((_R, D), lambda i: (i, 0)),
        pl.BlockSpec((D, D), lambda i: (0, 0)),
        pl.BlockSpec((1, D), lambda i: (0, 0)),
    ],
    out_specs=pl.BlockSpec((_R, D), lambda i: (i, 0)),
    out_shape=jax.ShapeDtypeStruct((N, D), jnp.float32),
)


def _make_tc_combine(relu):
    return pl.pallas_call(
        functools.partial(_tc_combine_body, relu),
        grid=(N // _R,),
        in_specs=[
            pl.BlockSpec((_R, DH), lambda i: (i, 0)),
            pl.BlockSpec((_R, DH), lambda i: (i, 0)),
            pl.BlockSpec((_R, 1), lambda i: (i, 0)),
            pl.BlockSpec((_R, D), lambda i: (i, 0)),
            pl.BlockSpec((D, D), lambda i: (0, 0)),
        ],
        out_specs=pl.BlockSpec((_R, D), lambda i: (i, 0)),
        out_shape=jax.ShapeDtypeStruct((N, D), jnp.float32),
    )


_tc_combine_relu = _make_tc_combine(True)
_tc_combine_last = _make_tc_combine(False)


def kernel(x, edge_index, Wl0, bl0, Wr0, Wl1, bl1, Wr1, Wl2, bl2, Wr2):
    src = edge_index[0]
    dst = edge_index[1]
    pad_src = (jnp.arange(EP - E, dtype=jnp.int32) % NB) * NPB
    src2d = jnp.concatenate([src, pad_src]).reshape(EP // 128, 128)
    dst2d = jnp.concatenate(
        [dst, jnp.full((EP - E,), N, jnp.int32)]).reshape(EP // 128, 128)
    z1d = jnp.zeros((ROWS_PER_TILE,), jnp.float32)
    z128 = jnp.zeros((ROWS_PER_TILE, DH), jnp.float32)

    junk_s = jnp.zeros((CAP,), jnp.int32)
    junk_d = jnp.full((CAP,), N, jnp.int32)
    psrc, pdst, degf = _partition(src2d, dst2d, z1d, junk_s, junk_d)
    psrc = psrc.reshape(NB, TILES, NCH, AC)
    pdst = pdst.reshape(NB, TILES, NCH, AC)
    deg2 = degf[:N].reshape(N, 1)

    h = x
    layers = [(Wl0, bl0, Wr0, True), (Wl1, bl1, Wr1, True),
              (Wl2, bl2, Wr2, False)]
    for Wl, bl, Wr, relu in layers:
        hp = jnp.pad(h, ((0, NP - N), (0, 0)))
        r = _tc_root(h, Wr, bl.reshape(1, D))
        a0, a1 = _sc_agg(hp[:, :DH], hp[:, DH:], psrc, pdst, z128)
        tc = _tc_combine_relu if relu else _tc_combine_last
        h = tc(a0, a1, deg2, r, Wl)
    return h
